# baseline (speedup 1.0000x reference)
import functools

import jax
import jax.numpy as jnp
from jax.experimental import pallas as pl
from jax.experimental.pallas import tpu as pltpu
from jax.experimental.pallas import tpu_sc as plsc

_E = 8
_BLK = 128
_T = 2048
_PADT = 2 * _T + _E * _BLK
_NB = _PADT // _BLK
_NW = 32
_TPW = _T // _NW


def _router_meta_body(xb_ref, gw_ref, dst_ref, g_ref, eob_ref, rank_ref,
                      ind_ref):
    xb = xb_ref[...]
    raw = jax.lax.dot_general(
        xb, gw_ref[...],
        dimension_numbers=(((1,), (1,)), ((), ())),
        preferred_element_type=jnp.float32)
    lsig = -jax.nn.softplus(-raw)
    z = lsig - jnp.max(lsig, axis=-1, keepdims=True)
    ez = jnp.exp(z)
    probs = ez / jnp.sum(ez, axis=-1, keepdims=True)
    i1 = jnp.argmax(probs, axis=-1, keepdims=True)
    cols = jax.lax.broadcasted_iota(jnp.int32, probs.shape, 1)
    m1 = jnp.max(probs, axis=-1, keepdims=True)
    masked = jnp.where(cols == i1, -jnp.inf, probs)
    m2 = jnp.max(masked, axis=-1, keepdims=True)
    i2 = jnp.argmax(masked, axis=-1, keepdims=True)
    denom = m1 + m2 + 1e-20
    g_ref[...] = jnp.concatenate([m1 / denom, m2 / denom], axis=1)

    sel = (cols == i1) | (cols == i2)
    ind = jnp.where(sel, 1.0, 0.0).astype(jnp.bfloat16)
    ind_ref[...] = ind
    T = ind.shape[0]
    cr = jax.lax.broadcasted_iota(jnp.int32, (128, 128), 0)
    cc = jax.lax.broadcasted_iota(jnp.int32, (128, 128), 1)
    tri = jnp.where(cr > cc, 1.0, 0.0).astype(jnp.bfloat16)

    def _chunk(c, carry):
        ind_c = ind_ref[pl.ds(c * 128, 128), :]
        rc = jnp.dot(tri, ind_c, preferred_element_type=jnp.float32) + carry
        rank_ref[pl.ds(c * 128, 128), :] = rc
        return carry + jnp.sum(ind_c.astype(jnp.float32), axis=0,
                               keepdims=True)

    counts_f = jax.lax.fori_loop(0, T // 128, _chunk,
                                 jnp.zeros((1, _E), jnp.float32))
    rank = rank_ref[...].astype(jnp.int32)
    counts = counts_f.astype(jnp.int32)
    nblk = (counts + (_BLK - 1)) >> 7
    er = jax.lax.broadcasted_iota(jnp.int32, (_E, _E), 0)
    ec = jax.lax.broadcasted_iota(jnp.int32, (_E, _E), 1)
    mlow = jnp.where(er < ec, 1.0, 0.0).astype(jnp.bfloat16)
    boff = jnp.dot(nblk.astype(jnp.bfloat16), mlow,
                   preferred_element_type=jnp.float32).astype(jnp.int32)
    pos = boff * _BLK + rank
    d1 = jnp.sum(jnp.where(cols == i1, pos, 0), axis=1, keepdims=True)
    d2 = jnp.sum(jnp.where(cols == i2, pos, 0), axis=1, keepdims=True)
    dst_ref[...] = jnp.concatenate([d1, d2], axis=1)

    bb = jax.lax.broadcasted_iota(jnp.int32, (_E, 128), 1)
    boffc = jnp.broadcast_to(boff.reshape(_E, 1), (_E, 128))
    eob_ref[...] = (jnp.sum(jnp.where(boffc <= bb, 1, 0), axis=0,
                            keepdims=True) - 1)


def _grouped_ffn_body(eob_ref, xs_ref, wu_ref, wd_ref, y_ref):
    del eob_ref
    xb = xs_ref[...].astype(jnp.bfloat16)
    up = jnp.dot(xb, wu_ref[0], preferred_element_type=jnp.float32)
    t = jnp.maximum(up, 0.0)
    act = (t * t).astype(jnp.bfloat16)
    y_ref[...] = jnp.dot(act, wd_ref[0], preferred_element_type=jnp.float32)


def _combine_body(y1_ref, y2_ref, g_ref, out_ref):
    g1 = g_ref[:, 0:1]
    g2 = g_ref[:, 1:2]
    out_ref[...] = y1_ref[...] * g1 + y2_ref[...] * g2


def _sc_scatter(x, d1, d2):
    mesh = plsc.VectorSubcoreMesh(core_axis_name="c", subcore_axis_name="s")

    @functools.partial(
        pl.kernel, mesh=mesh,
        out_type=jax.ShapeDtypeStruct((_PADT, 1024), jnp.float32),
        scratch_types=[
            pltpu.VMEM((_TPW,), jnp.int32),
            pltpu.VMEM((_TPW, 1024), jnp.float32),
            pltpu.SemaphoreType.DMA,
        ],
    )
    def k(x_hbm, d1_hbm, d2_hbm, xs_hbm, idx_v, rows_v, sem):
        wid = jax.lax.axis_index("s") * 2 + jax.lax.axis_index("c")
        base = wid * _TPW
        pltpu.sync_copy(x_hbm.at[pl.ds(base, _TPW)], rows_v)
        pltpu.sync_copy(d1_hbm.at[wid], idx_v)
        pltpu.async_copy(rows_v, xs_hbm.at[idx_v], sem).wait()
        pltpu.sync_copy(d2_hbm.at[wid], idx_v)
        pltpu.async_copy(rows_v, xs_hbm.at[idx_v], sem).wait()

    return k(x, d1, d2)


def _sc_gather(ys, d1, d2):
    mesh = plsc.VectorSubcoreMesh(core_axis_name="c", subcore_axis_name="s")

    @functools.partial(
        pl.kernel, mesh=mesh,
        out_type=(jax.ShapeDtypeStruct((_T, 1024), jnp.float32),
                  jax.ShapeDtypeStruct((_T, 1024), jnp.float32)),
        scratch_types=[
            pltpu.VMEM((_TPW,), jnp.int32),
            pltpu.VMEM((_TPW, 1024), jnp.float32),
            pltpu.SemaphoreType.DMA,
        ],
    )
    def k(ys_hbm, d1_hbm, d2_hbm, y1_hbm, y2_hbm, idx_v, rows_v, sem):
        wid = jax.lax.axis_index("s") * 2 + jax.lax.axis_index("c")
        base = wid * _TPW
        pltpu.sync_copy(d1_hbm.at[wid], idx_v)
        pltpu.async_copy(ys_hbm.at[idx_v], rows_v, sem).wait()
        pltpu.sync_copy(rows_v, y1_hbm.at[pl.ds(base, _TPW)])
        pltpu.sync_copy(d2_hbm.at[wid], idx_v)
        pltpu.async_copy(ys_hbm.at[idx_v], rows_v, sem).wait()
        pltpu.sync_copy(rows_v, y2_hbm.at[pl.ds(base, _TPW)])

    return k(ys, d1, d2)


def kernel(hidden_states, gate_weight, w_up, w_down):
    B, S, H = hidden_states.shape
    T = B * S
    I = w_up.shape[-1]
    x = hidden_states.reshape(T, H).astype(jnp.float32)
    xb = x.astype(jnp.bfloat16)
    gw = gate_weight.astype(jnp.bfloat16)
    wu = w_up.astype(jnp.bfloat16)
    wd = w_down.astype(jnp.bfloat16)

    dst, g12, eobp = pl.pallas_call(
        _router_meta_body,
        grid=(1,),
        in_specs=[
            pl.BlockSpec((T, H), lambda i: (0, 0)),
            pl.BlockSpec((_E, H), lambda i: (0, 0)),
        ],
        out_specs=[
            pl.BlockSpec((T, 2), lambda i: (0, 0)),
            pl.BlockSpec((T, 2), lambda i: (0, 0)),
            pl.BlockSpec((1, 128), lambda i: (0, 0)),
        ],
        out_shape=[
            jax.ShapeDtypeStruct((T, 2), jnp.int32),
            jax.ShapeDtypeStruct((T, 2), jnp.float32),
            jax.ShapeDtypeStruct((1, 128), jnp.int32),
        ],
        scratch_shapes=[pltpu.VMEM((T, _E), jnp.float32),
                        pltpu.VMEM((T, _E), jnp.bfloat16)],
    )(xb, gw)

    d1 = dst[:, 0].reshape(_NW, _TPW)
    d2 = dst[:, 1].reshape(_NW, _TPW)
    eob = eobp[0, :_NB]

    xs = _sc_scatter(x, d1, d2)

    ys = pl.pallas_call(
        _grouped_ffn_body,
        grid_spec=pltpu.PrefetchScalarGridSpec(
            num_scalar_prefetch=1,
            grid=(_NB,),
            in_specs=[
                pl.BlockSpec((_BLK, H), lambda i, eob_ref: (i, 0)),
                pl.BlockSpec((1, H, I), lambda i, eob_ref: (eob_ref[i], 0, 0)),
                pl.BlockSpec((1, I, H), lambda i, eob_ref: (eob_ref[i], 0, 0)),
            ],
            out_specs=pl.BlockSpec((_BLK, H), lambda i, eob_ref: (i, 0)),
        ),
        out_shape=jax.ShapeDtypeStruct((_PADT, H), jnp.float32),
        compiler_params=pltpu.CompilerParams(
            dimension_semantics=("arbitrary",)),
    )(eob, xs, wu, wd)

    y1, y2 = _sc_gather(ys, d1, d2)

    out = pl.pallas_call(
        _combine_body,
        grid=(1,),
        in_specs=[
            pl.BlockSpec((T, H), lambda i: (0, 0)),
            pl.BlockSpec((T, H), lambda i: (0, 0)),
            pl.BlockSpec((T, 2), lambda i: (0, 0)),
        ],
        out_specs=pl.BlockSpec((T, H), lambda i: (0, 0)),
        out_shape=jax.ShapeDtypeStruct((T, H), jnp.float32),
    )(y1, y2, g12)
    return out.reshape(B, S, H)


# speedup vs baseline: 1.0304x; 1.0304x over previous
import functools

import jax
import jax.numpy as jnp
from jax.experimental import pallas as pl
from jax.experimental.pallas import tpu as pltpu
from jax.experimental.pallas import tpu_sc as plsc

_E = 8
_BLK = 128
_T = 2048
_H = 1024
_PADT = 2 * _T + _E * _BLK
_NB = _PADT // _BLK
_NW = 32
_TPW = _T // _NW


def _router_meta_body(xb_ref, gw_ref, dst_ref, gb1_ref, gb2_ref, eob_ref,
                      rank_ref, ind_ref):
    xb = xb_ref[...]
    raw = jax.lax.dot_general(
        xb, gw_ref[...],
        dimension_numbers=(((1,), (1,)), ((), ())),
        preferred_element_type=jnp.float32)
    lsig = -jax.nn.softplus(-raw)
    z = lsig - jnp.max(lsig, axis=-1, keepdims=True)
    ez = jnp.exp(z)
    probs = ez / jnp.sum(ez, axis=-1, keepdims=True)
    i1 = jnp.argmax(probs, axis=-1, keepdims=True)
    cols = jax.lax.broadcasted_iota(jnp.int32, probs.shape, 1)
    m1 = jnp.max(probs, axis=-1, keepdims=True)
    masked = jnp.where(cols == i1, -jnp.inf, probs)
    m2 = jnp.max(masked, axis=-1, keepdims=True)
    i2 = jnp.argmax(masked, axis=-1, keepdims=True)
    denom = m1 + m2 + 1e-20
    gb1_ref[...] = jnp.broadcast_to(m1 / denom, gb1_ref.shape)
    gb2_ref[...] = jnp.broadcast_to(m2 / denom, gb2_ref.shape)

    sel = (cols == i1) | (cols == i2)
    ind = jnp.where(sel, 1.0, 0.0).astype(jnp.bfloat16)
    ind_ref[...] = ind
    T = ind.shape[0]
    cr = jax.lax.broadcasted_iota(jnp.int32, (128, 128), 0)
    cc = jax.lax.broadcasted_iota(jnp.int32, (128, 128), 1)
    tri = jnp.where(cr > cc, 1.0, 0.0).astype(jnp.bfloat16)

    def _chunk(c, carry):
        ind_c = ind_ref[pl.ds(c * 128, 128), :]
        rc = jnp.dot(tri, ind_c, preferred_element_type=jnp.float32) + carry
        rank_ref[pl.ds(c * 128, 128), :] = rc
        return carry + jnp.sum(ind_c.astype(jnp.float32), axis=0,
                               keepdims=True)

    counts_f = jax.lax.fori_loop(0, T // 128, _chunk,
                                 jnp.zeros((1, _E), jnp.float32))
    rank = rank_ref[...].astype(jnp.int32)
    counts = counts_f.astype(jnp.int32)
    nblk = (counts + (_BLK - 1)) >> 7
    er = jax.lax.broadcasted_iota(jnp.int32, (_E, _E), 0)
    ec = jax.lax.broadcasted_iota(jnp.int32, (_E, _E), 1)
    mlow = jnp.where(er < ec, 1.0, 0.0).astype(jnp.bfloat16)
    boff = jnp.dot(nblk.astype(jnp.bfloat16), mlow,
                   preferred_element_type=jnp.float32).astype(jnp.int32)
    pos = boff * _BLK + rank
    d1 = jnp.sum(jnp.where(cols == i1, pos, 0), axis=1, keepdims=True)
    d2 = jnp.sum(jnp.where(cols == i2, pos, 0), axis=1, keepdims=True)
    dst_ref[...] = jnp.concatenate([d1, d2], axis=1)

    bb = jax.lax.broadcasted_iota(jnp.int32, (_E, 128), 1)
    boffc = jnp.broadcast_to(boff.reshape(_E, 1), (_E, 128))
    eob_ref[...] = (jnp.sum(jnp.where(boffc <= bb, 1, 0), axis=0,
                            keepdims=True) - 1)


def _grouped_ffn_body(eob_ref, xs_ref, wu_ref, wd_ref, gs_ref, y_ref):
    del eob_ref
    xb = xs_ref[...].astype(jnp.bfloat16)
    up = jnp.dot(xb, wu_ref[0], preferred_element_type=jnp.float32)
    t = jnp.maximum(up, 0.0)
    act = (t * t).astype(jnp.bfloat16)
    y = jnp.dot(act, wd_ref[0], preferred_element_type=jnp.float32)
    y_ref[...] = y * gs_ref[:, 0:1]


def _sc_scatter(x, gb1, gb2, d1, d2):
    mesh = plsc.VectorSubcoreMesh(core_axis_name="c", subcore_axis_name="s")

    @functools.partial(
        pl.kernel, mesh=mesh,
        out_type=(jax.ShapeDtypeStruct((_PADT, _H), jnp.float32),
                  jax.ShapeDtypeStruct((_PADT, 128), jnp.float32)),
        scratch_types=[
            pltpu.VMEM((_TPW,), jnp.int32),
            pltpu.VMEM((_TPW,), jnp.int32),
            pltpu.VMEM((_TPW, _H), jnp.float32),
            pltpu.VMEM((_TPW, 128), jnp.float32),
            pltpu.VMEM((_TPW, 128), jnp.float32),
            pltpu.SemaphoreType.DMA,
        ],
    )
    def k(x_hbm, gb1_hbm, gb2_hbm, d1_hbm, d2_hbm, xs_hbm, gs_hbm,
          i1_v, i2_v, rows_v, g1_v, g2_v, sem):
        wid = jax.lax.axis_index("s") * 2 + jax.lax.axis_index("c")
        base = wid * _TPW
        pltpu.sync_copy(x_hbm.at[pl.ds(base, _TPW)], rows_v)
        pltpu.sync_copy(gb1_hbm.at[pl.ds(base, _TPW)], g1_v)
        pltpu.sync_copy(gb2_hbm.at[pl.ds(base, _TPW)], g2_v)
        pltpu.sync_copy(d1_hbm.at[wid], i1_v)
        pltpu.sync_copy(d2_hbm.at[wid], i2_v)
        c1 = pltpu.async_copy(rows_v, xs_hbm.at[i1_v], sem)
        c2 = pltpu.async_copy(rows_v, xs_hbm.at[i2_v], sem)
        c3 = pltpu.async_copy(g1_v, gs_hbm.at[i1_v], sem)
        c4 = pltpu.async_copy(g2_v, gs_hbm.at[i2_v], sem)
        c1.wait(); c2.wait(); c3.wait(); c4.wait()

    return k(x, gb1, gb2, d1, d2)


def _sc_gather_add(ys, d1, d2):
    mesh = plsc.VectorSubcoreMesh(core_axis_name="c", subcore_axis_name="s")
    _C = 32

    @functools.partial(
        pl.kernel, mesh=mesh,
        out_type=jax.ShapeDtypeStruct((_T, _H), jnp.float32),
        scratch_types=[
            pltpu.VMEM((_TPW,), jnp.int32),
            pltpu.VMEM((_TPW,), jnp.int32),
            pltpu.VMEM((_C, _H), jnp.float32),
            pltpu.VMEM((_C, _H), jnp.float32),
            pltpu.SemaphoreType.DMA,
            pltpu.SemaphoreType.DMA,
        ],
    )
    def k(ys_hbm, d1_hbm, d2_hbm, out_hbm, i1_v, i2_v, r1_v, r2_v, s1, s2):
        wid = jax.lax.axis_index("s") * 2 + jax.lax.axis_index("c")
        base = wid * _TPW
        pltpu.sync_copy(d1_hbm.at[wid], i1_v)
        pltpu.sync_copy(d2_hbm.at[wid], i2_v)
        for c in range(_TPW // _C):
            c1 = pltpu.async_copy(ys_hbm.at[i1_v.at[pl.ds(c * _C, _C)]],
                                  r1_v, s1)
            c2 = pltpu.async_copy(ys_hbm.at[i2_v.at[pl.ds(c * _C, _C)]],
                                  r2_v, s2)
            c1.wait(); c2.wait()

            def _tok(i, _):
                for j in range(_H // 16):
                    sl = pl.ds(j * 16, 16)
                    r1_v[i, sl] = r1_v[i, sl] + r2_v[i, sl]
                return 0

            jax.lax.fori_loop(0, _C, _tok, 0)
            pltpu.sync_copy(r1_v, out_hbm.at[pl.ds(base + c * _C, _C)])

    return k(ys, d1, d2)


def kernel(hidden_states, gate_weight, w_up, w_down):
    B, S, H = hidden_states.shape
    T = B * S
    I = w_up.shape[-1]
    x = hidden_states.reshape(T, H).astype(jnp.float32)
    xb = x.astype(jnp.bfloat16)
    gw = gate_weight.astype(jnp.bfloat16)
    wu = w_up.astype(jnp.bfloat16)
    wd = w_down.astype(jnp.bfloat16)

    dst, gb1, gb2, eobp = pl.pallas_call(
        _router_meta_body,
        grid=(1,),
        in_specs=[
            pl.BlockSpec((T, H), lambda i: (0, 0)),
            pl.BlockSpec((_E, H), lambda i: (0, 0)),
        ],
        out_specs=[
            pl.BlockSpec((T, 2), lambda i: (0, 0)),
            pl.BlockSpec((T, 128), lambda i: (0, 0)),
            pl.BlockSpec((T, 128), lambda i: (0, 0)),
            pl.BlockSpec((1, 128), lambda i: (0, 0)),
        ],
        out_shape=[
            jax.ShapeDtypeStruct((T, 2), jnp.int32),
            jax.ShapeDtypeStruct((T, 128), jnp.float32),
            jax.ShapeDtypeStruct((T, 128), jnp.float32),
            jax.ShapeDtypeStruct((1, 128), jnp.int32),
        ],
        scratch_shapes=[pltpu.VMEM((T, _E), jnp.float32),
                        pltpu.VMEM((T, _E), jnp.bfloat16)],
    )(xb, gw)

    d1 = dst[:, 0].reshape(_NW, _TPW)
    d2 = dst[:, 1].reshape(_NW, _TPW)
    eob = eobp[0, :_NB]

    xs, gs = _sc_scatter(x, gb1, gb2, d1, d2)

    ys = pl.pallas_call(
        _grouped_ffn_body,
        grid_spec=pltpu.PrefetchScalarGridSpec(
            num_scalar_prefetch=1,
            grid=(_NB,),
            in_specs=[
                pl.BlockSpec((_BLK, H), lambda i, eob_ref: (i, 0)),
                pl.BlockSpec((1, H, I), lambda i, eob_ref: (eob_ref[i], 0, 0)),
                pl.BlockSpec((1, I, H), lambda i, eob_ref: (eob_ref[i], 0, 0)),
                pl.BlockSpec((_BLK, 128), lambda i, eob_ref: (i, 0)),
            ],
            out_specs=pl.BlockSpec((_BLK, H), lambda i, eob_ref: (i, 0)),
        ),
        out_shape=jax.ShapeDtypeStruct((_PADT, H), jnp.float32),
        compiler_params=pltpu.CompilerParams(
            dimension_semantics=("arbitrary",)),
    )(eob, xs, wu, wd, gs)

    out = _sc_gather_add(ys, d1, d2)
    return out.reshape(B, S, H)


# speedup vs baseline: 1.5815x; 1.5349x over previous
import jax
import jax.numpy as jnp
from jax.experimental import pallas as pl
from jax.experimental.pallas import tpu as pltpu

_NUM_EXPERTS = 8


def _moe_body(gw_ref, x_ref, wu_ref, wd_ref, out_ref, gates_ref):
    e = pl.program_id(0)
    xb = x_ref[...]

    @pl.when(e == 0)
    def _router():
        raw = jax.lax.dot_general(
            xb, gw_ref[...],
            dimension_numbers=(((1,), (1,)), ((), ())),
            preferred_element_type=jnp.float32)
        lsig = -jax.nn.softplus(-raw)
        z = lsig - jnp.max(lsig, axis=-1, keepdims=True)
        ez = jnp.exp(z)
        probs = ez / jnp.sum(ez, axis=-1, keepdims=True)
        i1 = jnp.argmax(probs, axis=-1, keepdims=True)
        cols = jax.lax.broadcasted_iota(jnp.int32, probs.shape, 1)
        m1 = jnp.max(probs, axis=-1, keepdims=True)
        masked = jnp.where(cols == i1, -jnp.inf, probs)
        m2 = jnp.max(masked, axis=-1, keepdims=True)
        i2 = jnp.argmax(masked, axis=-1, keepdims=True)
        denom = m1 + m2 + 1e-20
        keep = (cols == i1) | (cols == i2)
        gates_ref[...] = jnp.where(keep, probs, 0.0) / denom

    gates = gates_ref[...]
    ecols = jax.lax.broadcasted_iota(jnp.int32, gates.shape, 1)
    g = jnp.sum(jnp.where(ecols == e, gates, 0.0), axis=1, keepdims=True)

    up = jnp.dot(xb, wu_ref[0].astype(jnp.bfloat16),
                 preferred_element_type=jnp.float32)
    t = jnp.maximum(up, 0.0)
    act = (t * t * g).astype(jnp.bfloat16)
    contrib = jnp.dot(act, wd_ref[0].astype(jnp.bfloat16),
                      preferred_element_type=jnp.float32)

    @pl.when(e == 0)
    def _init():
        out_ref[...] = contrib

    @pl.when(e != 0)
    def _acc():
        out_ref[...] += contrib


def kernel(hidden_states, gate_weight, w_up, w_down):
    B, S, H = hidden_states.shape
    T = B * S
    E = _NUM_EXPERTS
    I = w_up.shape[-1]
    x = hidden_states.reshape(T, H).astype(jnp.bfloat16)
    gw = gate_weight.astype(jnp.bfloat16)
    wu = w_up
    wd = w_down

    out = pl.pallas_call(
        _moe_body,
        grid=(E,),
        in_specs=[
            pl.BlockSpec((E, H), lambda e: (0, 0)),
            pl.BlockSpec((T, H), lambda e: (0, 0)),
            pl.BlockSpec((1, H, I), lambda e: (e, 0, 0)),
            pl.BlockSpec((1, I, H), lambda e: (e, 0, 0)),
        ],
        out_specs=pl.BlockSpec((T, H), lambda e: (0, 0)),
        out_shape=jax.ShapeDtypeStruct((T, H), jnp.float32),
        scratch_shapes=[pltpu.VMEM((T, E), jnp.float32)],
        compiler_params=pltpu.CompilerParams(
            dimension_semantics=("arbitrary",)),
    )(gw, x, wu, wd)
    return out.reshape(B, S, H)
